# baseline (speedup 1.0000x reference)
import functools

import jax
import jax.numpy as jnp
from jax import lax
from jax.experimental import pallas as pl
from jax.experimental.pallas import tpu as pltpu
from jax.experimental.pallas import tpu_sc as plsc

_B = 16384
_D = 128
_NW = 32
_BH = _B // 2
_BPW = _BH // _NW
_NCH = _BPW // 128
_BM = 4096


def _prep_body(ac_ref, bt_ref, rt_ref, at_ref, wo_ref, tj_ref):
    f32 = jnp.float32
    t_ac = jnp.dot(ac_ref[...], wo_ref[pl.ds(0, 32), :], preferred_element_type=f32)
    t_bt = jnp.dot(bt_ref[...], wo_ref[pl.ds(32, 32), :], preferred_element_type=f32)
    t_rt = jnp.dot(rt_ref[...], wo_ref[pl.ds(64, 16), :], preferred_element_type=f32)
    t_at = jnp.dot(at_ref[...], wo_ref[pl.ds(80, 16), :], preferred_element_type=f32)
    j = lax.broadcasted_iota(jnp.int32, (96, 1), 0)
    oh_a = (j // 24 == lax.broadcasted_iota(jnp.int32, (96, 4), 1)).astype(f32)
    oh_b = ((j // 6) % 4 == lax.broadcasted_iota(jnp.int32, (96, 4), 1)).astype(f32)
    oh_r = ((j // 3) % 2 == lax.broadcasted_iota(jnp.int32, (96, 2), 1)).astype(f32)
    oh_t = (j % 3 == lax.broadcasted_iota(jnp.int32, (96, 3), 1)).astype(f32)
    tj = (
        jnp.dot(oh_a, t_ac, preferred_element_type=f32)
        + jnp.dot(oh_b, t_bt, preferred_element_type=f32)
        + jnp.dot(oh_r, t_rt, preferred_element_type=f32)
        + jnp.dot(oh_t, t_at, preferred_element_type=f32)
    )
    for w in range(_NW):
        tj_ref[pl.ds(w * 96, 96), :] = tj


def _build_tj(ac_table, bt_table, rt_table, at_table, Wo):
    return pl.pallas_call(
        _prep_body,
        out_shape=jax.ShapeDtypeStruct((_NW * 96, _D), jnp.float32),
    )(ac_table, bt_table, rt_table, at_table, Wo)


def _sc_gather_body(ac_hbm, bt_hbm, rt_hbm, at_hbm, tj_hbm, out_hbm,
                    a_v, b_v, r_v, t_v, j0_v, j1_v, rows_v,
                    sem_i, sem_g, sem_o):
    nc = 2
    wid = lax.axis_index("s") * nc + lax.axis_index("c")
    base = wid * _BPW
    cp0 = pltpu.async_copy(ac_hbm.at[pl.ds(base, _BPW)], a_v, sem_i)
    cp1 = pltpu.async_copy(bt_hbm.at[pl.ds(base, _BPW)], b_v, sem_i)
    cp2 = pltpu.async_copy(rt_hbm.at[pl.ds(base, _BPW)], r_v, sem_i)
    cp3 = pltpu.async_copy(at_hbm.at[pl.ds(base, _BPW)], t_v, sem_i)
    cp0.wait(); cp1.wait(); cp2.wait(); cp3.wait()
    j_bufs = [j0_v, j1_v]
    tbase = wid * 96
    for k in range(_BPW // 16):
        sl = pl.ds(k * 16, 16)
        j = ((a_v[sl] * 4 + b_v[sl]) * 2 + r_v[sl]) * 3 + t_v[sl] + tbase
        j_bufs[k // 8][pl.ds((k % 8) * 16, 16)] = j
    gathers = [
        pltpu.async_copy(tj_hbm.at[j_bufs[c]], rows_v.at[pl.ds(c * 128, 128)], sem_g)
        for c in range(_NCH)
    ]
    outs = []
    for c in range(_NCH):
        gathers[c].wait()
        outs.append(pltpu.async_copy(
            rows_v.at[pl.ds(c * 128, 128)],
            out_hbm.at[pl.ds(base + c * 128, 128)], sem_o))
    for cp in outs:
        cp.wait()


def _sc_gather(asset_class, borrower_type, rate_type, amort_type, tj):
    mesh = plsc.VectorSubcoreMesh(core_axis_name="c", subcore_axis_name="s")
    run = functools.partial(
        pl.kernel,
        mesh=mesh,
        out_type=jax.ShapeDtypeStruct((_BH, _D), jnp.float32),
        scratch_types=[
            pltpu.VMEM((_BPW,), jnp.int32),
            pltpu.VMEM((_BPW,), jnp.int32),
            pltpu.VMEM((_BPW,), jnp.int32),
            pltpu.VMEM((_BPW,), jnp.int32),
            pltpu.VMEM((128,), jnp.int32),
            pltpu.VMEM((128,), jnp.int32),
            pltpu.VMEM((_BPW, _D), jnp.float32),
            pltpu.SemaphoreType.DMA,
            pltpu.SemaphoreType.DMA,
            pltpu.SemaphoreType.DMA,
        ],
    )(_sc_gather_body)
    return run(asset_class, borrower_type, rate_type, amort_type, tj)


def _main_body(x_ref, part_ref, w1_ref, b1_ref, w2_ref, b2_ref, wo_ref, bo_ref,
               out_ref):
    f32 = jnp.float32
    h = jnp.maximum(
        jnp.dot(x_ref[...], w1_ref[...], preferred_element_type=f32) + b1_ref[...],
        0.0,
    )
    wo5 = wo_ref[pl.ds(96, 32), :]
    w2o = jnp.dot(w2_ref[...], wo5, preferred_element_type=f32)
    cvec = jnp.dot(b2_ref[...], wo5, preferred_element_type=f32) + bo_ref[...]
    out_ref[...] = jnp.dot(h, w2o, preferred_element_type=f32) + part_ref[...] + cvec


def _main_body_alias(prev_ref, x_ref, part_ref, w1_ref, b1_ref, w2_ref, b2_ref,
                     wo_ref, bo_ref, out_ref):
    del prev_ref
    _main_body(x_ref, part_ref, w1_ref, b1_ref, w2_ref, b2_ref, wo_ref, bo_ref,
               out_ref)


_W_SPECS = [
    pl.BlockSpec((12, 64), lambda i: (0, 0)),
    pl.BlockSpec((1, 64), lambda i: (0, 0)),
    pl.BlockSpec((64, 32), lambda i: (0, 0)),
    pl.BlockSpec((1, 32), lambda i: (0, 0)),
    pl.BlockSpec((_D, _D), lambda i: (0, 0)),
    pl.BlockSpec((1, _D), lambda i: (0, 0)),
]
_NBLK_H = _BH // _BM


def _main_h0(x, part0, *weights):
    return pl.pallas_call(
        _main_body,
        grid=(_NBLK_H,),
        in_specs=[
            pl.BlockSpec((_BM, 12), lambda i: (i, 0)),
            pl.BlockSpec((_BM, _D), lambda i: (i, 0)),
        ] + _W_SPECS,
        out_specs=pl.BlockSpec((_BM, _D), lambda i: (i, 0)),
        out_shape=jax.ShapeDtypeStruct((_B, _D), jnp.float32),
    )(x, part0, *weights)


def _main_h1(prev, x, part1, *weights):
    return pl.pallas_call(
        _main_body_alias,
        grid=(_NBLK_H,),
        in_specs=[
            pl.BlockSpec((8, _D), lambda i: (0, 0)),
            pl.BlockSpec((_BM, 12), lambda i: (i + _NBLK_H, 0)),
            pl.BlockSpec((_BM, _D), lambda i: (i, 0)),
        ] + _W_SPECS,
        out_specs=pl.BlockSpec((_BM, _D), lambda i: (i + _NBLK_H, 0)),
        out_shape=jax.ShapeDtypeStruct((_B, _D), jnp.float32),
        input_output_aliases={0: 0},
    )(prev, x, part1, *weights)


def kernel(asset_class, borrower_type, rate_type, amort_type, continuous_features,
           ac_table, bt_table, rt_table, at_table, W1, b1, W2, b2, Wo, bo):
    tj = _build_tj(ac_table, bt_table, rt_table, at_table, Wo)
    part0 = _sc_gather(asset_class[:_BH], borrower_type[:_BH],
                       rate_type[:_BH], amort_type[:_BH], tj)
    part1 = _sc_gather(asset_class[_BH:], borrower_type[_BH:],
                       rate_type[_BH:], amort_type[_BH:], tj)
    weights = (W1, b1.reshape(1, 64), W2, b2.reshape(1, 32), Wo, bo.reshape(1, _D))
    out = _main_h0(continuous_features, part0, *weights)
    return _main_h1(out, continuous_features, part1, *weights)


# speedup vs baseline: 1.1238x; 1.1238x over previous
import functools

import jax
import jax.numpy as jnp
from jax import lax
from jax.experimental import pallas as pl
from jax.experimental.pallas import tpu as pltpu
from jax.experimental.pallas import tpu_sc as plsc

_B = 16384
_D = 128
_NW = 32
_BPW = _B // _NW
_NCH = _BPW // 128
_BM = 8192


def _prep_body(ac_ref, bt_ref, rt_ref, at_ref, wo_ref, tj_ref):
    f32 = jnp.float32
    t_ac = jnp.dot(ac_ref[...], wo_ref[pl.ds(0, 32), :], preferred_element_type=f32)
    t_bt = jnp.dot(bt_ref[...], wo_ref[pl.ds(32, 32), :], preferred_element_type=f32)
    t_rt = jnp.dot(rt_ref[...], wo_ref[pl.ds(64, 16), :], preferred_element_type=f32)
    t_at = jnp.dot(at_ref[...], wo_ref[pl.ds(80, 16), :], preferred_element_type=f32)
    j = lax.broadcasted_iota(jnp.int32, (96, 1), 0)
    oh_a = (j // 24 == lax.broadcasted_iota(jnp.int32, (96, 4), 1)).astype(f32)
    oh_b = ((j // 6) % 4 == lax.broadcasted_iota(jnp.int32, (96, 4), 1)).astype(f32)
    oh_r = ((j // 3) % 2 == lax.broadcasted_iota(jnp.int32, (96, 2), 1)).astype(f32)
    oh_t = (j % 3 == lax.broadcasted_iota(jnp.int32, (96, 3), 1)).astype(f32)
    tj = (
        jnp.dot(oh_a, t_ac, preferred_element_type=f32)
        + jnp.dot(oh_b, t_bt, preferred_element_type=f32)
        + jnp.dot(oh_r, t_rt, preferred_element_type=f32)
        + jnp.dot(oh_t, t_at, preferred_element_type=f32)
    )
    for w in range(_NW):
        tj_ref[pl.ds(w * 96, 96), :] = tj


def _build_tj(ac_table, bt_table, rt_table, at_table, Wo):
    return pl.pallas_call(
        _prep_body,
        out_shape=jax.ShapeDtypeStruct((_NW * 96, _D), jnp.float32),
    )(ac_table, bt_table, rt_table, at_table, Wo)


def _sc_gather_body(ac_hbm, bt_hbm, rt_hbm, at_hbm, tj_hbm, out_hbm,
                    a_v, b_v, r_v, t_v, j0_v, j1_v, j2_v, j3_v, rows_v,
                    sem_i, sem_g, sem_o):
    nc = 2
    wid = lax.axis_index("s") * nc + lax.axis_index("c")
    base = wid * _BPW
    cp0 = pltpu.async_copy(ac_hbm.at[pl.ds(base, _BPW)], a_v, sem_i)
    cp1 = pltpu.async_copy(bt_hbm.at[pl.ds(base, _BPW)], b_v, sem_i)
    cp2 = pltpu.async_copy(rt_hbm.at[pl.ds(base, _BPW)], r_v, sem_i)
    cp3 = pltpu.async_copy(at_hbm.at[pl.ds(base, _BPW)], t_v, sem_i)
    cp0.wait(); cp1.wait(); cp2.wait(); cp3.wait()
    j_bufs = [j0_v, j1_v, j2_v, j3_v]
    tbase = wid * 96
    for k in range(_BPW // 16):
        sl = pl.ds(k * 16, 16)
        j = ((a_v[sl] * 4 + b_v[sl]) * 2 + r_v[sl]) * 3 + t_v[sl] + tbase
        j_bufs[k // 8][pl.ds((k % 8) * 16, 16)] = j
    gathers = [
        pltpu.async_copy(tj_hbm.at[j_bufs[c]], rows_v.at[pl.ds(c * 128, 128)], sem_g)
        for c in range(_NCH)
    ]
    outs = []
    for c in range(_NCH):
        gathers[c].wait()
        outs.append(pltpu.async_copy(
            rows_v.at[pl.ds(c * 128, 128)],
            out_hbm.at[pl.ds(base + c * 128, 128)], sem_o))
    for cp in outs:
        cp.wait()


def _sc_gather(asset_class, borrower_type, rate_type, amort_type, tj):
    mesh = plsc.VectorSubcoreMesh(core_axis_name="c", subcore_axis_name="s")
    run = functools.partial(
        pl.kernel,
        mesh=mesh,
        out_type=jax.ShapeDtypeStruct((_B, _D), jnp.float32),
        scratch_types=[
            pltpu.VMEM((_BPW,), jnp.int32),
            pltpu.VMEM((_BPW,), jnp.int32),
            pltpu.VMEM((_BPW,), jnp.int32),
            pltpu.VMEM((_BPW,), jnp.int32),
            pltpu.VMEM((128,), jnp.int32),
            pltpu.VMEM((128,), jnp.int32),
            pltpu.VMEM((128,), jnp.int32),
            pltpu.VMEM((128,), jnp.int32),
            pltpu.VMEM((_BPW, _D), jnp.float32),
            pltpu.SemaphoreType.DMA,
            pltpu.SemaphoreType.DMA,
            pltpu.SemaphoreType.DMA,
        ],
    )(_sc_gather_body)
    return run(asset_class, borrower_type, rate_type, amort_type, tj)


def _main_body(x_ref, part_ref, w1_ref, b1_ref, w2_ref, b2_ref, wo_ref, bo_ref,
               out_ref):
    f32 = jnp.float32
    h = jnp.maximum(
        jnp.dot(x_ref[...], w1_ref[...], preferred_element_type=f32) + b1_ref[...],
        0.0,
    )
    wo5 = wo_ref[pl.ds(96, 32), :]
    w2o = jnp.dot(w2_ref[...], wo5, preferred_element_type=f32)
    cvec = jnp.dot(b2_ref[...], wo5, preferred_element_type=f32) + bo_ref[...]
    out_ref[...] = jnp.dot(h, w2o, preferred_element_type=f32) + part_ref[...] + cvec


def _main(x, part, W1, b1, W2, b2, Wo, bo):
    return pl.pallas_call(
        _main_body,
        grid=(_B // _BM,),
        in_specs=[
            pl.BlockSpec((_BM, 12), lambda i: (i, 0)),
            pl.BlockSpec((_BM, _D), lambda i: (i, 0)),
            pl.BlockSpec((12, 64), lambda i: (0, 0)),
            pl.BlockSpec((1, 64), lambda i: (0, 0)),
            pl.BlockSpec((64, 32), lambda i: (0, 0)),
            pl.BlockSpec((1, 32), lambda i: (0, 0)),
            pl.BlockSpec((_D, _D), lambda i: (0, 0)),
            pl.BlockSpec((1, _D), lambda i: (0, 0)),
        ],
        out_specs=pl.BlockSpec((_BM, _D), lambda i: (i, 0)),
        out_shape=jax.ShapeDtypeStruct((_B, _D), jnp.float32),
    )(x, part, W1, b1, W2, b2, Wo, bo)


def kernel(asset_class, borrower_type, rate_type, amort_type, continuous_features,
           ac_table, bt_table, rt_table, at_table, W1, b1, W2, b2, Wo, bo):
    tj = _build_tj(ac_table, bt_table, rt_table, at_table, Wo)
    part = _sc_gather(asset_class, borrower_type, rate_type, amort_type, tj)
    return _main(
        continuous_features, part,
        W1, b1.reshape(1, 64), W2, b2.reshape(1, 32), Wo, bo.reshape(1, _D),
    )
